# Initial kernel scaffold; baseline (speedup 1.0000x reference)
import jax
import jax.numpy as jnp
from jax.experimental import pallas as pl


def kernel(input_vec, word_embedding):
    raise NotImplementedError("write your pallas kernel here")


# speedup vs baseline: 1.8716x; 1.8716x over previous
import functools

import jax
import jax.numpy as jnp
from jax import lax
from jax.experimental import pallas as pl
from jax.experimental.pallas import tpu as pltpu
from jax.experimental.pallas import tpu_sc as plsc

VOCAB = 1000000
EMBED_DIM = 64
BATCH = 16384
HIST = 50

_NC = 2
_NS = 16
_NW = _NC * _NS

_TOTAL = BATCH * HIST
_PER_W = _TOTAL // _NW
_CHUNK = 128
_GROUP = 512
_CPG = _GROUP // _CHUNK
_NG = _PER_W // _GROUP
_NBUF = 2


def _body(idx_hbm, tab_hbm, out_hbm, idx_v, rows0, rows1, gsem0, gsem1,
          ssem0, ssem1):
    wid = lax.axis_index("s") * _NC + lax.axis_index("c")
    base = wid * _PER_W
    pltpu.sync_copy(idx_hbm.at[pl.ds(base, _PER_W)], idx_v)

    rows = (rows0, rows1)
    gsems = (gsem0, gsem1)
    ssems = (ssem0, ssem1)

    def step(t, _):
        for slot in range(_NBUF):
            g = t * _NBUF + slot
            @pl.when(t > 0)
            def _():
                pltpu.make_async_copy(
                    rows[slot], out_hbm.at[pl.ds(base, _GROUP)],
                    ssems[slot]).wait()

            descs = []
            for j in range(_CPG):
                descs.append(pltpu.async_copy(
                    tab_hbm.at[idx_v.at[pl.ds(g * _GROUP + j * _CHUNK,
                                              _CHUNK)]],
                    rows[slot].at[pl.ds(j * _CHUNK, _CHUNK)],
                    gsems[slot]))
            for d in descs:
                d.wait()
            pltpu.async_copy(
                rows[slot], out_hbm.at[pl.ds(base + g * _GROUP, _GROUP)],
                ssems[slot])
        return ()

    lax.fori_loop(0, _NG // _NBUF, step, (), unroll=False)
    for slot in range(_NBUF):
        pltpu.make_async_copy(
            rows[slot], out_hbm.at[pl.ds(base, _GROUP)], ssems[slot]).wait()


@jax.jit
def _lookup(idx_flat, word_embedding):
    mesh = plsc.VectorSubcoreMesh(core_axis_name="c", subcore_axis_name="s")
    fn = pl.kernel(
        _body,
        out_type=jax.ShapeDtypeStruct((_TOTAL, EMBED_DIM), jnp.float32),
        mesh=mesh,
        scratch_types=[
            pltpu.VMEM((_PER_W,), jnp.int32),
            pltpu.VMEM((_GROUP, EMBED_DIM), jnp.float32),
            pltpu.VMEM((_GROUP, EMBED_DIM), jnp.float32),
            pltpu.SemaphoreType.DMA,
            pltpu.SemaphoreType.DMA,
            pltpu.SemaphoreType.DMA,
            pltpu.SemaphoreType.DMA,
        ],
        compiler_params=pltpu.CompilerParams(use_tc_tiling_on_sc=False),
    )
    return fn(idx_flat, word_embedding)


def kernel(input_vec, word_embedding):
    idx_flat = input_vec.reshape(-1).astype(jnp.int32)
    out = _lookup(idx_flat, word_embedding)
    return out.reshape(BATCH, HIST, EMBED_DIM)


# speedup vs baseline: 1.8756x; 1.0022x over previous
import functools

import jax
import jax.numpy as jnp
from jax import lax
from jax.experimental import pallas as pl
from jax.experimental.pallas import tpu as pltpu
from jax.experimental.pallas import tpu_sc as plsc

VOCAB = 1000000
EMBED_DIM = 64
BATCH = 16384
HIST = 50

_NC = 2
_NS = 16
_NW = _NC * _NS

_TOTAL = BATCH * HIST
_PER_W = _TOTAL // _NW
_CHUNK = 128
_GROUP = 256
_CPG = _GROUP // _CHUNK
_NG = _PER_W // _GROUP
_NBUF = 5
_AHEAD = 4


def _gather_group(tab_hbm, idx_v, rows, sem, g):
    for j in range(_CPG):
        pltpu.async_copy(
            tab_hbm.at[idx_v.at[pl.ds(g * _GROUP + j * _CHUNK, _CHUNK)]],
            rows.at[pl.ds(j * _CHUNK, _CHUNK)],
            sem)


def _drain_group(tab_hbm, rows, sem):
    pltpu.make_async_copy(tab_hbm.at[pl.ds(0, _GROUP)], rows, sem).wait()


def _body(idx_hbm, tab_hbm, out_hbm, idx_v,
          rows0, rows1, rows2, rows3, rows4,
          gsem0, gsem1, gsem2, gsem3, gsem4,
          ssem0, ssem1, ssem2, ssem3, ssem4):
    wid = lax.axis_index("s") * _NC + lax.axis_index("c")
    base = wid * _PER_W
    pltpu.sync_copy(idx_hbm.at[pl.ds(base, _PER_W)], idx_v)

    rows = (rows0, rows1, rows2, rows3, rows4)
    gsems = (gsem0, gsem1, gsem2, gsem3, gsem4)
    ssems = (ssem0, ssem1, ssem2, ssem3, ssem4)

    def store_done_wait(slot):
        pltpu.make_async_copy(
            rows[slot], out_hbm.at[pl.ds(base, _GROUP)], ssems[slot]).wait()

    def retire(slot, g):
        _drain_group(tab_hbm, rows[slot], gsems[slot])
        pltpu.async_copy(
            rows[slot], out_hbm.at[pl.ds(base + g * _GROUP, _GROUP)],
            ssems[slot])

    def step(t, _):
        for slot in range(_NBUF):
            g = t * _NBUF + slot
            @pl.when(t > 0)
            def _():
                store_done_wait(slot)

            _gather_group(tab_hbm, idx_v, rows[slot], gsems[slot], g)

            ps = (slot - _AHEAD) % _NBUF
            if slot >= _AHEAD:
                retire(ps, g - _AHEAD)
            else:
                @pl.when(t > 0)
                def _():
                    retire(ps, g - _AHEAD)
        return ()

    lax.fori_loop(0, _NG // _NBUF, step, (), unroll=False)
    for g in range(_NG - _AHEAD, _NG):
        retire(g % _NBUF, g)
    for slot in range(_NBUF):
        store_done_wait(slot)


@jax.jit
def _lookup(idx_flat, word_embedding):
    mesh = plsc.VectorSubcoreMesh(core_axis_name="c", subcore_axis_name="s")
    fn = pl.kernel(
        _body,
        out_type=jax.ShapeDtypeStruct((_TOTAL, EMBED_DIM), jnp.float32),
        mesh=mesh,
        scratch_types=(
            [pltpu.VMEM((_PER_W,), jnp.int32)]
            + [pltpu.VMEM((_GROUP, EMBED_DIM), jnp.float32)] * _NBUF
            + [pltpu.SemaphoreType.DMA] * (2 * _NBUF)
        ),
        compiler_params=pltpu.CompilerParams(use_tc_tiling_on_sc=False),
    )
    return fn(idx_flat, word_embedding)


def kernel(input_vec, word_embedding):
    idx_flat = input_vec.reshape(-1).astype(jnp.int32)
    out = _lookup(idx_flat, word_embedding)
    return out.reshape(BATCH, HIST, EMBED_DIM)
